# baseline (speedup 1.0000x reference)
import functools

import jax
import jax.numpy as jnp
from jax import lax
from jax.experimental import pallas as pl
from jax.experimental.pallas import tpu as pltpu
from jax.experimental.pallas import tpu_sc as plsc

NUM_EMB = 1000000
D = 64
B = 4096 * 200
NC = 2
NS = 16
NW = NC * NS
B_PER_W = B // NW
SUB = 128
CHUNK = 512
N_SUB = CHUNK // SUB
N_CHUNK = B_PER_W // CHUNK
ROWS_PER_CHUNK_2D = CHUNK // SUB


def _make_kernel():
  mesh = plsc.VectorSubcoreMesh(core_axis_name="c", subcore_axis_name="s")

  @functools.partial(
      pl.kernel,
      mesh=mesh,
      out_type=jax.ShapeDtypeStruct((B, D), jnp.float32),
      compiler_params=pltpu.CompilerParams(use_tc_tiling_on_sc=False),
      scratch_types=[
          pltpu.VMEM((N_SUB, SUB), jnp.int32),
          pltpu.VMEM((CHUNK, D), jnp.float32),
          pltpu.SemaphoreType.DMA,
      ],
  )
  def emb_kernel(idx_hbm, table_hbm, out_hbm, idx_v, rows_v, sem):
    cid = lax.axis_index("c")
    sid = lax.axis_index("s")
    wid = sid * NC + cid
    row_base = wid * (B_PER_W // SUB)
    out_base = wid * B_PER_W

    def body(ci, _):
      pltpu.sync_copy(
          idx_hbm.at[pl.ds(row_base + ci * ROWS_PER_CHUNK_2D, ROWS_PER_CHUNK_2D)],
          idx_v)
      copies = []
      for j in range(N_SUB):
        copies.append(
            pltpu.async_copy(
                table_hbm.at[idx_v.at[j]],
                rows_v.at[pl.ds(j * SUB, SUB)],
                sem))
      for c in copies:
        c.wait()
      pltpu.sync_copy(rows_v, out_hbm.at[pl.ds(out_base + ci * CHUNK, CHUNK)])
      return 0

    lax.fori_loop(0, N_CHUNK, body, 0)

  return emb_kernel


_EMB_KERNEL = _make_kernel()


@jax.jit
def kernel(token_ids, W):
  idx2d = token_ids.reshape(B // SUB, SUB).astype(jnp.int32)
  out = _EMB_KERNEL(idx2d, W)
  return out.reshape(token_ids.shape + (D,))


# speedup vs baseline: 1.0439x; 1.0439x over previous
import functools

import jax
import jax.numpy as jnp
from jax import lax
from jax.experimental import pallas as pl
from jax.experimental.pallas import tpu as pltpu
from jax.experimental.pallas import tpu_sc as plsc

NUM_EMB = 1000000
D = 64
B = 4096 * 200
NC = 2
NS = 16
NW = NC * NS
B_PER_W = B // NW
SUB = 128
CHUNK = 512
N_SUB = CHUNK // SUB
N_CHUNK = B_PER_W // CHUNK
IDX_ROWS = CHUNK // SUB


def _make_kernel():
  mesh = plsc.VectorSubcoreMesh(core_axis_name="c", subcore_axis_name="s")

  @functools.partial(
      pl.kernel,
      mesh=mesh,
      out_type=jax.ShapeDtypeStruct((B, D), jnp.float32),
      compiler_params=pltpu.CompilerParams(use_tc_tiling_on_sc=False),
      scratch_types=[
          pltpu.VMEM((2, N_SUB, SUB), jnp.int32),
          pltpu.VMEM((CHUNK, D), jnp.float32),
          pltpu.VMEM((CHUNK, D), jnp.float32),
          pltpu.SemaphoreType.DMA,
          pltpu.SemaphoreType.DMA,
          pltpu.SemaphoreType.DMA,
          pltpu.SemaphoreType.DMA,
          pltpu.SemaphoreType.DMA,
      ],
  )
  def emb_kernel(idx_hbm, table_hbm, out_hbm, idx_v, rows_v0, rows_v1,
                 idx_sem, g_sem0, g_sem1, wb_sem0, wb_sem1):
    cid = lax.axis_index("c")
    sid = lax.axis_index("s")
    wid = sid * NC + cid
    row_base = wid * (B_PER_W // SUB)
    out_base = wid * B_PER_W

    rows = (rows_v0, rows_v1)
    g_sems = (g_sem0, g_sem1)
    wb_sems = (wb_sem0, wb_sem1)

    pltpu.async_copy(idx_hbm.at[pl.ds(row_base, IDX_ROWS)], idx_v.at[0], idx_sem)

    def outer(gi, _):
      for b in range(2):
        ci = gi * 2 + b
        pltpu.make_async_copy(
            idx_hbm.at[pl.ds(0, IDX_ROWS)], idx_v.at[b], idx_sem).wait()
        @pl.when(ci >= 2)
        def _():
          pltpu.make_async_copy(
              out_hbm.at[pl.ds(0, CHUNK)], rows[b], wb_sems[b]).wait()
        copies = []
        for j in range(N_SUB):
          copies.append(
              pltpu.async_copy(
                  table_hbm.at[idx_v.at[b].at[j]],
                  rows[b].at[pl.ds(j * SUB, SUB)],
                  g_sems[b]))
        @pl.when(ci + 1 < N_CHUNK)
        def _():
          pltpu.async_copy(
              idx_hbm.at[pl.ds(row_base + (ci + 1) * IDX_ROWS, IDX_ROWS)],
              idx_v.at[1 - b], idx_sem)
        for c in copies:
          c.wait()
        pltpu.async_copy(
            rows[b], out_hbm.at[pl.ds(out_base + ci * CHUNK, CHUNK)], wb_sems[b])
      return 0

    lax.fori_loop(0, N_CHUNK // 2, outer, 0)
    for b in range(2):
      pltpu.make_async_copy(
          out_hbm.at[pl.ds(0, CHUNK)], rows[b], wb_sems[b]).wait()

  return emb_kernel


_EMB_KERNEL = _make_kernel()


@jax.jit
def kernel(token_ids, W):
  idx2d = token_ids.reshape(B // SUB, SUB).astype(jnp.int32)
  out = _EMB_KERNEL(idx2d, W)
  return out.reshape(token_ids.shape + (D,))
